# Initial kernel scaffold; baseline (speedup 1.0000x reference)
import jax
import jax.numpy as jnp
from jax.experimental import pallas as pl


def kernel(x, edge_index, batch, W1, b1, W2, b2, W3, b3, LW1, Lb1, LW2, Lb2):
    raise NotImplementedError("write your pallas kernel here")


# speedup vs baseline: 80.6606x; 80.6606x over previous
import jax
import jax.numpy as jnp
from jax import lax
from jax.experimental import pallas as pl
from jax.experimental.pallas import tpu as pltpu
from jax.experimental.pallas import tpu_sc as plsc

N = 10000
E = 320000
G = 16
F = 128
NC = 2
NS = 16
NW = NC * NS
L = 16
EB = 128
NB = 80
EPW = NB * EB
EPAD = NW * EPW
NP = 10240
RPT = NP // NS
R = 2048
PREC = lax.Precision.HIGHEST
f32 = jnp.float32

_MESH = plsc.VectorSubcoreMesh(core_axis_name="c", subcore_axis_name="s")


def _hist_body(dst_hbm, zeros_hbm, cnt_hbm, idx_v, ones_v, acc):
    c = lax.axis_index("c")
    s = lax.axis_index("s")
    wid = c * NS + s
    pltpu.sync_copy(dst_hbm.at[wid], idx_v)
    pltpu.sync_copy(zeros_hbm, acc.at[pl.ds(s * RPT, RPT)])

    def fill(j, carry):
        for k in range(F // L):
            ones_v[j, pl.ds(k * L, L)] = jnp.full((L,), 1.0, f32)
        return carry

    lax.fori_loop(jnp.int32(0), jnp.int32(EB), fill, jnp.int32(0))
    plsc.subcore_barrier()

    def body(j, carry):
        pltpu.sync_copy(ones_v, acc.at[idx_v.at[j]], add=True)
        return carry

    lax.fori_loop(jnp.int32(0), jnp.int32(NB), body, jnp.int32(0))
    plsc.subcore_barrier()
    pltpu.sync_copy(acc.at[pl.ds(s * RPT, RPT)],
                    cnt_hbm.at[c, pl.ds(s * RPT, RPT)])


_sc_hist = pl.kernel(
    _hist_body,
    out_type=jax.ShapeDtypeStruct((NC, NP, F), f32),
    mesh=_MESH,
    scratch_types=[
        pltpu.VMEM((NB, EB), jnp.int32),
        pltpu.VMEM((EB, F), f32),
        pltpu.VMEM_SHARED((NP, F), f32),
    ],
)


def _edge_body(y_hbm, src_hbm, dst_hbm, zeros_hbm, out_hbm,
               srcv, dstv, rows, acc, sem):
    c = lax.axis_index("c")
    s = lax.axis_index("s")
    wid = c * NS + s
    pltpu.sync_copy(src_hbm.at[wid], srcv)
    pltpu.sync_copy(dst_hbm.at[wid], dstv)
    pltpu.sync_copy(zeros_hbm, acc.at[pl.ds(s * RPT, RPT)])
    plsc.subcore_barrier()

    def body(j, carry):
        pltpu.async_copy(y_hbm.at[srcv.at[j]], rows, sem).wait()
        pltpu.sync_copy(rows, acc.at[dstv.at[j]], add=True)
        return carry

    lax.fori_loop(jnp.int32(0), jnp.int32(NB), body, jnp.int32(0))
    plsc.subcore_barrier()
    pltpu.sync_copy(acc.at[pl.ds(s * RPT, RPT)],
                    out_hbm.at[c, pl.ds(s * RPT, RPT)])


_sc_edge = pl.kernel(
    _edge_body,
    out_type=jax.ShapeDtypeStruct((NC, NP, F), f32),
    mesh=_MESH,
    scratch_types=[
        pltpu.VMEM((NB, EB), jnp.int32),
        pltpu.VMEM((NB, EB), jnp.int32),
        pltpu.VMEM((EB, F), f32),
        pltpu.VMEM_SHARED((NP, F), f32),
        pltpu.SemaphoreType.DMA,
    ],
)


def _z(i):
    return i - i

def _mm1_body(x_ref, w_ref, cnt_ref, y_ref, dinv_ref):
    cnt = cnt_ref[...]
    deg = 1.0 + cnt[0, :, 0] + cnt[1, :, 0]
    dinv = lax.rsqrt(deg)
    z = jnp.dot(x_ref[...], w_ref[...], preferred_element_type=f32,
                precision=PREC)
    y_ref[...] = z * dinv[:, None]
    dinv_ref[...] = dinv[:, None]


def _tc_mm1(x, w, cnt):
    return pl.pallas_call(
        _mm1_body,
        grid=(NP // R,),
        in_specs=[
            pl.BlockSpec((R, F), lambda i: (i, _z(i))),
            pl.BlockSpec((F, F), lambda i: (_z(i), _z(i))),
            pl.BlockSpec((NC, R, F), lambda i: (_z(i), i, _z(i))),
        ],
        out_specs=[
            pl.BlockSpec((R, F), lambda i: (i, _z(i))),
            pl.BlockSpec((R, 1), lambda i: (i, _z(i))),
        ],
        out_shape=[
            jax.ShapeDtypeStruct((NP, F), f32),
            jax.ShapeDtypeStruct((NP, 1), f32),
        ],
    )(x, w, cnt)


def _comb_body_noprev(s_ref, y_ref, dinv_ref, b_ref, w_ref, h_ref, yn_ref):
    sarr = s_ref[...]
    dinv = dinv_ref[...]
    h = jnp.maximum(dinv * (sarr[0] + sarr[1] + y_ref[...]) + b_ref[...], 0.0)
    h_ref[...] = h
    yn_ref[...] = jnp.dot(h, w_ref[...], preferred_element_type=f32,
                          precision=PREC) * dinv


def _comb_body_prev(s_ref, y_ref, dinv_ref, b_ref, w_ref, prev_ref,
                    h_ref, yn_ref):
    sarr = s_ref[...]
    dinv = dinv_ref[...]
    h = jnp.maximum(dinv * (sarr[0] + sarr[1] + y_ref[...]) + b_ref[...], 0.0)
    h = h + prev_ref[...]
    h_ref[...] = h
    yn_ref[...] = jnp.dot(h, w_ref[...], preferred_element_type=f32,
                          precision=PREC) * dinv


def _tc_comb(s, y, dinv, b, w, prev=None):
    in_specs = [
        pl.BlockSpec((NC, R, F), lambda i: (_z(i), i, _z(i))),
        pl.BlockSpec((R, F), lambda i: (i, _z(i))),
        pl.BlockSpec((R, 1), lambda i: (i, _z(i))),
        pl.BlockSpec((1, F), lambda i: (_z(i), _z(i))),
        pl.BlockSpec((F, F), lambda i: (_z(i), _z(i))),
    ]
    args = [s, y, dinv, b, w]
    body = _comb_body_noprev
    if prev is not None:
        in_specs.append(pl.BlockSpec((R, F), lambda i: (i, _z(i))))
        args.append(prev)
        body = _comb_body_prev
    return pl.pallas_call(
        body,
        grid=(NP // R,),
        in_specs=in_specs,
        out_specs=[
            pl.BlockSpec((R, F), lambda i: (i, _z(i))),
            pl.BlockSpec((R, F), lambda i: (i, _z(i))),
        ],
        out_shape=[
            jax.ShapeDtypeStruct((NP, F), f32),
            jax.ShapeDtypeStruct((NP, F), f32),
        ],
    )(*args)


def _final_body(s_ref, y_ref, dinv_ref, b_ref, prev_ref, batch_ref,
                lw1_ref, lb1_ref, lw2_ref, lb2_ref, out_ref, sums, cnts):
    i = pl.program_id(0)

    @pl.when(i == 0)
    def _():
        sums[...] = jnp.zeros_like(sums)
        cnts[...] = jnp.zeros_like(cnts)

    sarr = s_ref[...]
    dinv = dinv_ref[...]
    h = jnp.maximum(dinv * (sarr[0] + sarr[1] + y_ref[...]) + b_ref[...], 0.0)
    h = h + prev_ref[...]
    onehot = (batch_ref[...] ==
              lax.broadcasted_iota(jnp.int32, (1, G), 1)).astype(f32)
    sums[...] += lax.dot_general(onehot, h, (((0,), (0,)), ((), ())),
                                 preferred_element_type=f32, precision=PREC)
    cnts[...] += jnp.sum(onehot, axis=0)[:, None]

    @pl.when(i == NP // R - 1)
    def _():
        g = sums[...] / jnp.maximum(cnts[...], 1.0)
        a = jnp.maximum(jnp.dot(g, lw1_ref[...], preferred_element_type=f32,
                                precision=PREC) + lb1_ref[...], 0.0)
        out_ref[...] = jnp.dot(a, lw2_ref[...], preferred_element_type=f32,
                               precision=PREC) + lb2_ref[...]


def _tc_final(s, y, dinv, b, prev, batch2d, lw1, lb1, lw2, lb2):
    return pl.pallas_call(
        _final_body,
        grid=(NP // R,),
        in_specs=[
            pl.BlockSpec((NC, R, F), lambda i: (_z(i), i, _z(i))),
            pl.BlockSpec((R, F), lambda i: (i, _z(i))),
            pl.BlockSpec((R, 1), lambda i: (i, _z(i))),
            pl.BlockSpec((1, F), lambda i: (_z(i), _z(i))),
            pl.BlockSpec((R, F), lambda i: (i, _z(i))),
            pl.BlockSpec((R, 1), lambda i: (i, _z(i))),
            pl.BlockSpec((F, F), lambda i: (_z(i), _z(i))),
            pl.BlockSpec((1, F), lambda i: (_z(i), _z(i))),
            pl.BlockSpec((F, F), lambda i: (_z(i), _z(i))),
            pl.BlockSpec((1, F), lambda i: (_z(i), _z(i))),
        ],
        out_specs=pl.BlockSpec((G, F), lambda i: (_z(i), _z(i))),
        out_shape=jax.ShapeDtypeStruct((G, F), f32),
        scratch_shapes=[
            pltpu.VMEM((G, F), f32),
            pltpu.VMEM((G, F), f32),
        ],
    )(s, y, dinv, b, prev, batch2d, lw1, lb1, lw2, lb2)


def kernel(x, edge_index, batch, W1, b1, W2, b2, W3, b3, LW1, Lb1, LW2, Lb2):
    x = x.astype(f32)
    src = edge_index[0].astype(jnp.int32)
    dst = edge_index[1].astype(jnp.int32)
    pad = jnp.full((EPAD - E,), N, jnp.int32)
    src_p = jnp.concatenate([src, pad]).reshape(NW, NB, EB)
    dst_p = jnp.concatenate([dst, pad]).reshape(NW, NB, EB)
    batch_p = jnp.concatenate(
        [batch.astype(jnp.int32), jnp.full((NP - N,), G, jnp.int32)]
    ).reshape(NP, 1)
    x_p = jnp.pad(x, ((0, NP - N), (0, 0)))

    zeros_acc = jnp.zeros((RPT, F), f32)

    cnt = _sc_hist(dst_p, zeros_acc)
    y1, dinv = _tc_mm1(x_p, W1.astype(f32), cnt)

    s1 = _sc_edge(y1, src_p, dst_p, zeros_acc)
    h1, y2 = _tc_comb(s1, y1, dinv, b1.reshape(1, F).astype(f32),
                      W2.astype(f32))

    s2 = _sc_edge(y2, src_p, dst_p, zeros_acc)
    h2, y3 = _tc_comb(s2, y2, dinv, b2.reshape(1, F).astype(f32),
                      W3.astype(f32), prev=h1)

    s3 = _sc_edge(y3, src_p, dst_p, zeros_acc)
    out = _tc_final(s3, y3, dinv, b3.reshape(1, F).astype(f32), h2,
                    batch_p, LW1.astype(f32), Lb1.reshape(1, F).astype(f32),
                    LW2.astype(f32), Lb2.reshape(1, F).astype(f32))
    return out.astype(jnp.float64)


# speedup vs baseline: 88.7484x; 1.1003x over previous
import jax
import jax.numpy as jnp
from jax import lax
from jax.experimental import pallas as pl
from jax.experimental.pallas import tpu as pltpu
from jax.experimental.pallas import tpu_sc as plsc

N = 10000
E = 320000
G = 16
F = 128
NC = 2
NS = 16
NW = NC * NS
L = 16
EB = 128
NB = 80
EPW = NB * EB
EPAD = NW * EPW
NP = 10240
RPT = NP // NS
R = 2048
PREC = lax.Precision.HIGHEST
f32 = jnp.float32

_MESH = plsc.VectorSubcoreMesh(core_axis_name="c", subcore_axis_name="s")


def _hist_body(dst_hbm, zeros_hbm, cnt_hbm, idx_v, ones_v, acc):
    c = lax.axis_index("c")
    s = lax.axis_index("s")
    wid = c * NS + s
    pltpu.sync_copy(dst_hbm.at[wid], idx_v)
    pltpu.sync_copy(zeros_hbm, acc.at[pl.ds(s * RPT, RPT)])

    def fill(j, carry):
        for k in range(F // L):
            ones_v[j, pl.ds(k * L, L)] = jnp.full((L,), 1.0, f32)
        return carry

    lax.fori_loop(jnp.int32(0), jnp.int32(EB), fill, jnp.int32(0))
    plsc.subcore_barrier()

    def body(j, carry):
        pltpu.sync_copy(ones_v, acc.at[idx_v.at[j]], add=True)
        return carry

    lax.fori_loop(jnp.int32(0), jnp.int32(NB), body, jnp.int32(0))
    plsc.subcore_barrier()
    pltpu.sync_copy(acc.at[pl.ds(s * RPT, RPT)],
                    cnt_hbm.at[c, pl.ds(s * RPT, RPT)])


_sc_hist = pl.kernel(
    _hist_body,
    out_type=jax.ShapeDtypeStruct((NC, NP, F), f32),
    mesh=_MESH,
    scratch_types=[
        pltpu.VMEM((NB, EB), jnp.int32),
        pltpu.VMEM((EB, F), f32),
        pltpu.VMEM_SHARED((NP, F), f32),
    ],
)


KBUF = 2
CHN = 5
CB = NB // CHN


def _edge_body(y_hbm, src_hbm, dst_hbm, zeros_hbm, out_hbm,
               srcv, dv0, dv1, b0, b1, acc, semg, semi):
    bufs = (b0, b1)
    dvs = (dv0, dv1)
    c = lax.axis_index("c")
    s = lax.axis_index("s")
    wid = c * NS + s
    pltpu.sync_copy(src_hbm.at[wid], srcv)
    pltpu.async_copy(dst_hbm.at[wid, pl.ds(0, CB)], dv0, semi)
    for r in range(KBUF):
        pltpu.async_copy(y_hbm.at[srcv.at[jnp.int32(r)]], bufs[r], semg)
    pltpu.sync_copy(zeros_hbm, acc.at[pl.ds(s * RPT, RPT)])
    plsc.subcore_barrier()

    for ci in range(CHN):
        dv = dvs[ci % 2]
        pltpu.make_async_copy(dst_hbm.at[wid, pl.ds(ci * CB, CB)],
                              dv, semi).wait()
        if ci + 1 < CHN:
            pltpu.async_copy(dst_hbm.at[wid, pl.ds((ci + 1) * CB, CB)],
                             dvs[(ci + 1) % 2], semi)
        ngroups = CB // KBUF if ci + 1 < CHN else CB // KBUF - 1

        def group(g, carry, _ci=ci, _dv=dv):
            for r in range(KBUF):
                j = _ci * CB + g * KBUF + r
                jl = g * KBUF + r
                pltpu.make_async_copy(y_hbm.at[srcv.at[j]],
                                      bufs[r], semg).wait()
                pltpu.sync_copy(bufs[r], acc.at[_dv.at[jl]], add=True)
                pltpu.async_copy(y_hbm.at[srcv.at[j + KBUF]], bufs[r], semg)
            return carry

        lax.fori_loop(jnp.int32(0), jnp.int32(ngroups), group, jnp.int32(0))
    for r in range(KBUF):
        j = jnp.int32(NB - KBUF + r)
        jl = jnp.int32(CB - KBUF + r)
        pltpu.make_async_copy(y_hbm.at[srcv.at[j]], bufs[r], semg).wait()
        pltpu.sync_copy(bufs[r], acc.at[dvs[(CHN - 1) % 2].at[jl]], add=True)
    plsc.subcore_barrier()
    pltpu.sync_copy(acc.at[pl.ds(s * RPT, RPT)],
                    out_hbm.at[c, pl.ds(s * RPT, RPT)])


_sc_edge = pl.kernel(
    _edge_body,
    out_type=jax.ShapeDtypeStruct((NC, NP, F), f32),
    mesh=_MESH,
    scratch_types=[
        pltpu.VMEM((NB, EB), jnp.int32),
        pltpu.VMEM((CB, EB), jnp.int32),
        pltpu.VMEM((CB, EB), jnp.int32),
        pltpu.VMEM((EB, F), f32),
        pltpu.VMEM((EB, F), f32),
        pltpu.VMEM_SHARED((NP, F), f32),
        pltpu.SemaphoreType.DMA,
        pltpu.SemaphoreType.DMA,
    ],
)


def _z(i):
    return i - i

def _mm1_body(x_ref, w_ref, cnt_ref, y_ref, dinv_ref):
    cnt = cnt_ref[...]
    deg = 1.0 + cnt[0, :, 0] + cnt[1, :, 0]
    dinv = lax.rsqrt(deg)
    z = jnp.dot(x_ref[...], w_ref[...], preferred_element_type=f32,
                precision=PREC)
    y_ref[...] = z * dinv[:, None]
    dinv_ref[...] = dinv[:, None]


def _tc_mm1(x, w, cnt):
    return pl.pallas_call(
        _mm1_body,
        grid=(NP // R,),
        in_specs=[
            pl.BlockSpec((R, F), lambda i: (i, _z(i))),
            pl.BlockSpec((F, F), lambda i: (_z(i), _z(i))),
            pl.BlockSpec((NC, R, F), lambda i: (_z(i), i, _z(i))),
        ],
        out_specs=[
            pl.BlockSpec((R, F), lambda i: (i, _z(i))),
            pl.BlockSpec((R, 1), lambda i: (i, _z(i))),
        ],
        out_shape=[
            jax.ShapeDtypeStruct((NP, F), f32),
            jax.ShapeDtypeStruct((NP, 1), f32),
        ],
    )(x, w, cnt)


def _comb_body_noprev(s_ref, y_ref, dinv_ref, b_ref, w_ref, h_ref, yn_ref):
    sarr = s_ref[...]
    dinv = dinv_ref[...]
    h = jnp.maximum(dinv * (sarr[0] + sarr[1] + y_ref[...]) + b_ref[...], 0.0)
    h_ref[...] = h
    yn_ref[...] = jnp.dot(h, w_ref[...], preferred_element_type=f32,
                          precision=PREC) * dinv


def _comb_body_prev(s_ref, y_ref, dinv_ref, b_ref, w_ref, prev_ref,
                    h_ref, yn_ref):
    sarr = s_ref[...]
    dinv = dinv_ref[...]
    h = jnp.maximum(dinv * (sarr[0] + sarr[1] + y_ref[...]) + b_ref[...], 0.0)
    h = h + prev_ref[...]
    h_ref[...] = h
    yn_ref[...] = jnp.dot(h, w_ref[...], preferred_element_type=f32,
                          precision=PREC) * dinv


def _tc_comb(s, y, dinv, b, w, prev=None):
    in_specs = [
        pl.BlockSpec((NC, R, F), lambda i: (_z(i), i, _z(i))),
        pl.BlockSpec((R, F), lambda i: (i, _z(i))),
        pl.BlockSpec((R, 1), lambda i: (i, _z(i))),
        pl.BlockSpec((1, F), lambda i: (_z(i), _z(i))),
        pl.BlockSpec((F, F), lambda i: (_z(i), _z(i))),
    ]
    args = [s, y, dinv, b, w]
    body = _comb_body_noprev
    if prev is not None:
        in_specs.append(pl.BlockSpec((R, F), lambda i: (i, _z(i))))
        args.append(prev)
        body = _comb_body_prev
    return pl.pallas_call(
        body,
        grid=(NP // R,),
        in_specs=in_specs,
        out_specs=[
            pl.BlockSpec((R, F), lambda i: (i, _z(i))),
            pl.BlockSpec((R, F), lambda i: (i, _z(i))),
        ],
        out_shape=[
            jax.ShapeDtypeStruct((NP, F), f32),
            jax.ShapeDtypeStruct((NP, F), f32),
        ],
    )(*args)


def _final_body(s_ref, y_ref, dinv_ref, b_ref, prev_ref, batch_ref,
                lw1_ref, lb1_ref, lw2_ref, lb2_ref, out_ref, sums, cnts):
    i = pl.program_id(0)

    @pl.when(i == 0)
    def _():
        sums[...] = jnp.zeros_like(sums)
        cnts[...] = jnp.zeros_like(cnts)

    sarr = s_ref[...]
    dinv = dinv_ref[...]
    h = jnp.maximum(dinv * (sarr[0] + sarr[1] + y_ref[...]) + b_ref[...], 0.0)
    h = h + prev_ref[...]
    onehot = (batch_ref[...] ==
              lax.broadcasted_iota(jnp.int32, (1, G), 1)).astype(f32)
    sums[...] += lax.dot_general(onehot, h, (((0,), (0,)), ((), ())),
                                 preferred_element_type=f32, precision=PREC)
    cnts[...] += jnp.sum(onehot, axis=0)[:, None]

    @pl.when(i == NP // R - 1)
    def _():
        g = sums[...] / jnp.maximum(cnts[...], 1.0)
        a = jnp.maximum(jnp.dot(g, lw1_ref[...], preferred_element_type=f32,
                                precision=PREC) + lb1_ref[...], 0.0)
        out_ref[...] = jnp.dot(a, lw2_ref[...], preferred_element_type=f32,
                               precision=PREC) + lb2_ref[...]


def _tc_final(s, y, dinv, b, prev, batch2d, lw1, lb1, lw2, lb2):
    return pl.pallas_call(
        _final_body,
        grid=(NP // R,),
        in_specs=[
            pl.BlockSpec((NC, R, F), lambda i: (_z(i), i, _z(i))),
            pl.BlockSpec((R, F), lambda i: (i, _z(i))),
            pl.BlockSpec((R, 1), lambda i: (i, _z(i))),
            pl.BlockSpec((1, F), lambda i: (_z(i), _z(i))),
            pl.BlockSpec((R, F), lambda i: (i, _z(i))),
            pl.BlockSpec((R, 1), lambda i: (i, _z(i))),
            pl.BlockSpec((F, F), lambda i: (_z(i), _z(i))),
            pl.BlockSpec((1, F), lambda i: (_z(i), _z(i))),
            pl.BlockSpec((F, F), lambda i: (_z(i), _z(i))),
            pl.BlockSpec((1, F), lambda i: (_z(i), _z(i))),
        ],
        out_specs=pl.BlockSpec((G, F), lambda i: (_z(i), _z(i))),
        out_shape=jax.ShapeDtypeStruct((G, F), f32),
        scratch_shapes=[
            pltpu.VMEM((G, F), f32),
            pltpu.VMEM((G, F), f32),
        ],
    )(s, y, dinv, b, prev, batch2d, lw1, lb1, lw2, lb2)


def kernel(x, edge_index, batch, W1, b1, W2, b2, W3, b3, LW1, Lb1, LW2, Lb2):
    x = x.astype(f32)
    src = edge_index[0].astype(jnp.int32)
    dst = edge_index[1].astype(jnp.int32)
    pad = jnp.full((EPAD - E,), N, jnp.int32)
    src_p = jnp.concatenate([src, pad]).reshape(NW, NB, EB)
    dst_p = jnp.concatenate([dst, pad]).reshape(NW, NB, EB)
    batch_p = jnp.concatenate(
        [batch.astype(jnp.int32), jnp.full((NP - N,), G, jnp.int32)]
    ).reshape(NP, 1)
    x_p = jnp.pad(x, ((0, NP - N), (0, 0)))

    zeros_acc = jnp.zeros((RPT, F), f32)

    cnt = _sc_hist(dst_p, zeros_acc)
    y1, dinv = _tc_mm1(x_p, W1.astype(f32), cnt)

    s1 = _sc_edge(y1, src_p, dst_p, zeros_acc)
    h1, y2 = _tc_comb(s1, y1, dinv, b1.reshape(1, F).astype(f32),
                      W2.astype(f32))

    s2 = _sc_edge(y2, src_p, dst_p, zeros_acc)
    h2, y3 = _tc_comb(s2, y2, dinv, b2.reshape(1, F).astype(f32),
                      W3.astype(f32), prev=h1)

    s3 = _sc_edge(y3, src_p, dst_p, zeros_acc)
    out = _tc_final(s3, y3, dinv, b3.reshape(1, F).astype(f32), h2,
                    batch_p, LW1.astype(f32), Lb1.reshape(1, F).astype(f32),
                    LW2.astype(f32), Lb2.reshape(1, F).astype(f32))
    return out.astype(jnp.float64)


# speedup vs baseline: 312.4383x; 3.5205x over previous
import jax
import jax.numpy as jnp
from jax import lax
from jax.experimental import pallas as pl
from jax.experimental.pallas import tpu as pltpu
from jax.experimental.pallas import tpu_sc as plsc

N = 10000
E = 320000
G = 16
F = 128
NC = 2
NS = 16
NW = NC * NS
L = 16
EB = 128
NB = 80
EPW = NB * EB
EPAD = NW * EPW
NP = 10240
RPT = NP // NS
R = 2048
PREC = lax.Precision.HIGHEST
f32 = jnp.float32

_MESH = plsc.VectorSubcoreMesh(core_axis_name="c", subcore_axis_name="s")


def _hist_body(dst_hbm, zeros_hbm, cnt_hbm, idx_v, ones_v, acc):
    c = lax.axis_index("c")
    s = lax.axis_index("s")
    wid = c * NS + s
    pltpu.sync_copy(dst_hbm.at[wid], idx_v)
    pltpu.sync_copy(zeros_hbm, acc.at[pl.ds(s * RPT, RPT)])

    def fill(j, carry):
        for k in range(F // L):
            ones_v[j, pl.ds(k * L, L)] = jnp.full((L,), 1.0, f32)
        return carry

    lax.fori_loop(jnp.int32(0), jnp.int32(EB), fill, jnp.int32(0))
    plsc.subcore_barrier()

    def body(j, carry):
        pltpu.sync_copy(ones_v, acc.at[idx_v.at[j]], add=True)
        return carry

    lax.fori_loop(jnp.int32(0), jnp.int32(NB), body, jnp.int32(0))
    plsc.subcore_barrier()
    pltpu.sync_copy(acc.at[pl.ds(s * RPT, RPT)],
                    cnt_hbm.at[c, pl.ds(s * RPT, RPT)])


_sc_hist = pl.kernel(
    _hist_body,
    out_type=jax.ShapeDtypeStruct((NC, NP, F), f32),
    mesh=_MESH,
    scratch_types=[
        pltpu.VMEM((NB, EB), jnp.int32),
        pltpu.VMEM((EB, F), f32),
        pltpu.VMEM_SHARED((NP, F), f32),
    ],
)


KBUF = 2
CHN = 5
CB = NB // CHN


def _edge_body(y_hbm, src_hbm, dst_hbm, zeros_hbm, out_hbm,
               srcv, dv0, dv1, b0, b1, acc, semg, semi):
    bufs = (b0, b1)
    dvs = (dv0, dv1)
    c = lax.axis_index("c")
    s = lax.axis_index("s")
    wid = c * NS + s
    pltpu.sync_copy(src_hbm.at[wid], srcv)
    pltpu.async_copy(dst_hbm.at[wid, pl.ds(0, CB)], dv0, semi)
    for r in range(KBUF):
        pltpu.async_copy(y_hbm.at[srcv.at[jnp.int32(r)]], bufs[r], semg)
    pltpu.sync_copy(zeros_hbm, acc.at[pl.ds(s * RPT, RPT)])
    plsc.subcore_barrier()

    for ci in range(CHN):
        dv = dvs[ci % 2]
        pltpu.make_async_copy(dst_hbm.at[wid, pl.ds(ci * CB, CB)],
                              dv, semi).wait()
        if ci + 1 < CHN:
            pltpu.async_copy(dst_hbm.at[wid, pl.ds((ci + 1) * CB, CB)],
                             dvs[(ci + 1) % 2], semi)
        ngroups = CB // KBUF if ci + 1 < CHN else CB // KBUF - 1

        def group(g, carry, _ci=ci, _dv=dv):
            for r in range(KBUF):
                j = _ci * CB + g * KBUF + r
                jl = g * KBUF + r
                pltpu.make_async_copy(y_hbm.at[srcv.at[j]],
                                      bufs[r], semg).wait()
                pltpu.sync_copy(bufs[r], acc.at[_dv.at[jl]], add=True)
                pltpu.async_copy(y_hbm.at[srcv.at[j + KBUF]], bufs[r], semg)
            return carry

        lax.fori_loop(jnp.int32(0), jnp.int32(ngroups), group, jnp.int32(0))
    for r in range(KBUF):
        j = jnp.int32(NB - KBUF + r)
        jl = jnp.int32(CB - KBUF + r)
        pltpu.make_async_copy(y_hbm.at[srcv.at[j]], bufs[r], semg).wait()
        pltpu.sync_copy(bufs[r], acc.at[dvs[(CHN - 1) % 2].at[jl]], add=True)
    plsc.subcore_barrier()
    pltpu.sync_copy(acc.at[pl.ds(s * RPT, RPT)],
                    out_hbm.at[c, pl.ds(s * RPT, RPT)])


_sc_edge = pl.kernel(
    _edge_body,
    out_type=jax.ShapeDtypeStruct((NC, NP, F), f32),
    mesh=_MESH,
    scratch_types=[
        pltpu.VMEM((NB, EB), jnp.int32),
        pltpu.VMEM((CB, EB), jnp.int32),
        pltpu.VMEM((CB, EB), jnp.int32),
        pltpu.VMEM((EB, F), f32),
        pltpu.VMEM((EB, F), f32),
        pltpu.VMEM_SHARED((NP, F), f32),
        pltpu.SemaphoreType.DMA,
        pltpu.SemaphoreType.DMA,
    ],
)


def _z(i):
    return i - i

def _mm1_body(x_ref, w_ref, cnt_ref, y_ref, dinv_ref):
    cnt = cnt_ref[...]
    deg = 1.0 + cnt[0, :, 0] + cnt[1, :, 0]
    dinv = lax.rsqrt(deg)
    z = jnp.dot(x_ref[...], w_ref[...], preferred_element_type=f32,
                precision=PREC)
    y_ref[...] = z * dinv[:, None]
    dinv_ref[...] = dinv[:, None]


def _tc_mm1(x, w, cnt):
    return pl.pallas_call(
        _mm1_body,
        grid=(NP // R,),
        in_specs=[
            pl.BlockSpec((R, F), lambda i: (i, _z(i))),
            pl.BlockSpec((F, F), lambda i: (_z(i), _z(i))),
            pl.BlockSpec((NC, R, F), lambda i: (_z(i), i, _z(i))),
        ],
        out_specs=[
            pl.BlockSpec((R, F), lambda i: (i, _z(i))),
            pl.BlockSpec((R, 1), lambda i: (i, _z(i))),
        ],
        out_shape=[
            jax.ShapeDtypeStruct((NP, F), f32),
            jax.ShapeDtypeStruct((NP, 1), f32),
        ],
    )(x, w, cnt)


def _valid_rows():
    rows = (lax.broadcasted_iota(jnp.int32, (R, 1), 0)
            + pl.program_id(0) * jnp.int32(R))
    return rows < jnp.int32(N)


def _comb_body_noprev(s_ref, y_ref, dinv_ref, b_ref, w_ref, h_ref, yn_ref):
    sarr = s_ref[...]
    dinv = dinv_ref[...]
    h = jnp.maximum(dinv * (sarr[0] + sarr[1] + y_ref[...]) + b_ref[...], 0.0)
    h_ref[...] = h
    yn = jnp.dot(h, w_ref[...], preferred_element_type=f32,
                 precision=PREC) * dinv
    yn_ref[...] = jnp.where(_valid_rows(), yn, 0.0)


def _comb_body_prev(s_ref, y_ref, dinv_ref, b_ref, w_ref, prev_ref,
                    h_ref, yn_ref):
    sarr = s_ref[...]
    dinv = dinv_ref[...]
    h = jnp.maximum(dinv * (sarr[0] + sarr[1] + y_ref[...]) + b_ref[...], 0.0)
    h = h + prev_ref[...]
    h_ref[...] = h
    yn = jnp.dot(h, w_ref[...], preferred_element_type=f32,
                 precision=PREC) * dinv
    yn_ref[...] = jnp.where(_valid_rows(), yn, 0.0)


def _tc_comb(s, y, dinv, b, w, prev=None):
    in_specs = [
        pl.BlockSpec((NC, R, F), lambda i: (_z(i), i, _z(i))),
        pl.BlockSpec((R, F), lambda i: (i, _z(i))),
        pl.BlockSpec((R, 1), lambda i: (i, _z(i))),
        pl.BlockSpec((1, F), lambda i: (_z(i), _z(i))),
        pl.BlockSpec((F, F), lambda i: (_z(i), _z(i))),
    ]
    args = [s, y, dinv, b, w]
    body = _comb_body_noprev
    if prev is not None:
        in_specs.append(pl.BlockSpec((R, F), lambda i: (i, _z(i))))
        args.append(prev)
        body = _comb_body_prev
    return pl.pallas_call(
        body,
        grid=(NP // R,),
        in_specs=in_specs,
        out_specs=[
            pl.BlockSpec((R, F), lambda i: (i, _z(i))),
            pl.BlockSpec((R, F), lambda i: (i, _z(i))),
        ],
        out_shape=[
            jax.ShapeDtypeStruct((NP, F), f32),
            jax.ShapeDtypeStruct((NP, F), f32),
        ],
    )(*args)


def _final_body(s_ref, y_ref, dinv_ref, b_ref, prev_ref, batch_ref,
                lw1_ref, lb1_ref, lw2_ref, lb2_ref, out_ref, sums, cnts):
    i = pl.program_id(0)

    @pl.when(i == 0)
    def _():
        sums[...] = jnp.zeros_like(sums)
        cnts[...] = jnp.zeros_like(cnts)

    sarr = s_ref[...]
    dinv = dinv_ref[...]
    h = jnp.maximum(dinv * (sarr[0] + sarr[1] + y_ref[...]) + b_ref[...], 0.0)
    h = h + prev_ref[...]
    onehot = (batch_ref[...] ==
              lax.broadcasted_iota(jnp.int32, (1, G), 1)).astype(f32)
    sums[...] += lax.dot_general(onehot, h, (((0,), (0,)), ((), ())),
                                 preferred_element_type=f32, precision=PREC)
    cnts[...] += jnp.sum(onehot, axis=0)[:, None]

    @pl.when(i == NP // R - 1)
    def _():
        g = sums[...] / jnp.maximum(cnts[...], 1.0)
        a = jnp.maximum(jnp.dot(g, lw1_ref[...], preferred_element_type=f32,
                                precision=PREC) + lb1_ref[...], 0.0)
        out_ref[...] = jnp.dot(a, lw2_ref[...], preferred_element_type=f32,
                               precision=PREC) + lb2_ref[...]


def _tc_final(s, y, dinv, b, prev, batch2d, lw1, lb1, lw2, lb2):
    return pl.pallas_call(
        _final_body,
        grid=(NP // R,),
        in_specs=[
            pl.BlockSpec((NC, R, F), lambda i: (_z(i), i, _z(i))),
            pl.BlockSpec((R, F), lambda i: (i, _z(i))),
            pl.BlockSpec((R, 1), lambda i: (i, _z(i))),
            pl.BlockSpec((1, F), lambda i: (_z(i), _z(i))),
            pl.BlockSpec((R, F), lambda i: (i, _z(i))),
            pl.BlockSpec((R, 1), lambda i: (i, _z(i))),
            pl.BlockSpec((F, F), lambda i: (_z(i), _z(i))),
            pl.BlockSpec((1, F), lambda i: (_z(i), _z(i))),
            pl.BlockSpec((F, F), lambda i: (_z(i), _z(i))),
            pl.BlockSpec((1, F), lambda i: (_z(i), _z(i))),
        ],
        out_specs=pl.BlockSpec((G, F), lambda i: (_z(i), _z(i))),
        out_shape=jax.ShapeDtypeStruct((G, F), f32),
        scratch_shapes=[
            pltpu.VMEM((G, F), f32),
            pltpu.VMEM((G, F), f32),
        ],
    )(s, y, dinv, b, prev, batch2d, lw1, lb1, lw2, lb2)


def kernel(x, edge_index, batch, W1, b1, W2, b2, W3, b3, LW1, Lb1, LW2, Lb2):
    x = x.astype(f32)
    src = edge_index[0].astype(jnp.int32)
    dst = edge_index[1].astype(jnp.int32)
    pad = N + jnp.arange(EPAD - E, dtype=jnp.int32) % jnp.int32(NP - N)
    src_p = jnp.concatenate([src, pad]).reshape(NW, NB, EB)
    dst_p = jnp.concatenate([dst, pad]).reshape(NW, NB, EB)
    batch_p = jnp.concatenate(
        [batch.astype(jnp.int32), jnp.full((NP - N,), G, jnp.int32)]
    ).reshape(NP, 1)
    x_p = jnp.pad(x, ((0, NP - N), (0, 0)))

    zeros_acc = jnp.zeros((RPT, F), f32)

    cnt = _sc_hist(dst_p, zeros_acc)
    y1, dinv = _tc_mm1(x_p, W1.astype(f32), cnt)

    s1 = _sc_edge(y1, src_p, dst_p, zeros_acc)
    h1, y2 = _tc_comb(s1, y1, dinv, b1.reshape(1, F).astype(f32),
                      W2.astype(f32))

    s2 = _sc_edge(y2, src_p, dst_p, zeros_acc)
    h2, y3 = _tc_comb(s2, y2, dinv, b2.reshape(1, F).astype(f32),
                      W3.astype(f32), prev=h1)

    s3 = _sc_edge(y3, src_p, dst_p, zeros_acc)
    out = _tc_final(s3, y3, dinv, b3.reshape(1, F).astype(f32), h2,
                    batch_p, LW1.astype(f32), Lb1.reshape(1, F).astype(f32),
                    LW2.astype(f32), Lb2.reshape(1, F).astype(f32))
    return out.astype(jnp.float64)


# speedup vs baseline: 312.9129x; 1.0015x over previous
import jax
import jax.numpy as jnp
from jax import lax
from jax.experimental import pallas as pl
from jax.experimental.pallas import tpu as pltpu
from jax.experimental.pallas import tpu_sc as plsc

N = 10000
E = 320000
G = 16
F = 128
NC = 2
NS = 16
NW = NC * NS
L = 16
EB = 128
NB = 80
EPW = NB * EB
EPAD = NW * EPW
NP = 10240
RPT = NP // NS
R = 2048
PREC = lax.Precision.HIGHEST
f32 = jnp.float32

_MESH = plsc.VectorSubcoreMesh(core_axis_name="c", subcore_axis_name="s")


def _hist_body(dst_hbm, zeros_hbm, cnt_hbm, idx_v, ones_v, acc):
    c = lax.axis_index("c")
    s = lax.axis_index("s")
    wid = c * NS + s
    pltpu.sync_copy(dst_hbm.at[wid], idx_v)
    pltpu.sync_copy(zeros_hbm, acc.at[pl.ds(s * RPT, RPT)])

    def fill(j, carry):
        for k in range(F // L):
            ones_v[j, pl.ds(k * L, L)] = jnp.full((L,), 1.0, f32)
        return carry

    lax.fori_loop(jnp.int32(0), jnp.int32(EB), fill, jnp.int32(0))
    plsc.subcore_barrier()

    def body(j, carry):
        pltpu.sync_copy(ones_v, acc.at[idx_v.at[j]], add=True)
        return carry

    lax.fori_loop(jnp.int32(0), jnp.int32(NB), body, jnp.int32(0))
    plsc.subcore_barrier()
    pltpu.sync_copy(acc.at[pl.ds(s * RPT, RPT)],
                    cnt_hbm.at[c, pl.ds(s * RPT, RPT)])


_sc_hist = pl.kernel(
    _hist_body,
    out_type=jax.ShapeDtypeStruct((NC, NP, F), f32),
    mesh=_MESH,
    scratch_types=[
        pltpu.VMEM((NB, EB), jnp.int32),
        pltpu.VMEM((EB, F), f32),
        pltpu.VMEM_SHARED((NP, F), f32),
    ],
)


KBUF = 2
CHN = 5
CB = NB // CHN


def _edge_body(y_hbm, src_hbm, dst_hbm, zeros_hbm, out_hbm,
               srcv, dv0, dv1, b0, b1, acc, semg, semi):
    bufs = (b0, b1)
    dvs = (dv0, dv1)
    c = lax.axis_index("c")
    s = lax.axis_index("s")
    wid = c * NS + s
    pltpu.sync_copy(src_hbm.at[wid], srcv)
    pltpu.async_copy(dst_hbm.at[wid, pl.ds(0, CB)], dv0, semi)
    for r in range(KBUF):
        pltpu.async_copy(y_hbm.at[srcv.at[jnp.int32(r)]], bufs[r], semg)
    pltpu.sync_copy(zeros_hbm, acc.at[pl.ds(s * RPT, RPT)])
    plsc.subcore_barrier()

    for ci in range(CHN):
        dv = dvs[ci % 2]
        pltpu.make_async_copy(dst_hbm.at[wid, pl.ds(ci * CB, CB)],
                              dv, semi).wait()
        if ci + 1 < CHN:
            pltpu.async_copy(dst_hbm.at[wid, pl.ds((ci + 1) * CB, CB)],
                             dvs[(ci + 1) % 2], semi)
        ngroups = CB // KBUF if ci + 1 < CHN else CB // KBUF - 1

        def group(g, carry, _ci=ci, _dv=dv):
            for r in range(KBUF):
                j = _ci * CB + g * KBUF + r
                jl = g * KBUF + r
                pltpu.make_async_copy(y_hbm.at[srcv.at[j]],
                                      bufs[r], semg).wait()
                pltpu.sync_copy(bufs[r], acc.at[_dv.at[jl]], add=True)
                pltpu.async_copy(y_hbm.at[srcv.at[j + KBUF]], bufs[r], semg)
            return carry

        lax.fori_loop(jnp.int32(0), jnp.int32(ngroups), group, jnp.int32(0))
    for r in range(KBUF):
        j = jnp.int32(NB - KBUF + r)
        jl = jnp.int32(CB - KBUF + r)
        pltpu.make_async_copy(y_hbm.at[srcv.at[j]], bufs[r], semg).wait()
        pltpu.sync_copy(bufs[r], acc.at[dvs[(CHN - 1) % 2].at[jl]], add=True)
    plsc.subcore_barrier()
    pltpu.sync_copy(acc.at[pl.ds(s * RPT, RPT)],
                    out_hbm.at[c, pl.ds(s * RPT, RPT)])


_sc_edge = pl.kernel(
    _edge_body,
    out_type=jax.ShapeDtypeStruct((NC, NP, F), f32),
    mesh=_MESH,
    scratch_types=[
        pltpu.VMEM((NB, EB), jnp.int32),
        pltpu.VMEM((CB, EB), jnp.int32),
        pltpu.VMEM((CB, EB), jnp.int32),
        pltpu.VMEM((EB, F), f32),
        pltpu.VMEM((EB, F), f32),
        pltpu.VMEM_SHARED((NP, F), f32),
        pltpu.SemaphoreType.DMA,
        pltpu.SemaphoreType.DMA,
    ],
)


def _z(i):
    return i - i

def _z_body(x_ref, w_ref, z_ref):
    z_ref[...] = jnp.dot(x_ref[...], w_ref[...], preferred_element_type=f32,
                         precision=PREC)


def _tc_z(x, w):
    return pl.pallas_call(
        _z_body,
        grid=(NP // R,),
        in_specs=[
            pl.BlockSpec((R, F), lambda i: (i, _z(i))),
            pl.BlockSpec((F, F), lambda i: (_z(i), _z(i))),
        ],
        out_specs=pl.BlockSpec((R, F), lambda i: (i, _z(i))),
        out_shape=jax.ShapeDtypeStruct((NP, F), f32),
    )(x, w)


def _scale_body(z_ref, cnt_ref, y_ref, dinv_ref):
    cnt = cnt_ref[...]
    deg = 1.0 + cnt[0, :, 0] + cnt[1, :, 0]
    dinv = lax.rsqrt(deg)
    y_ref[...] = z_ref[...] * dinv[:, None]
    dinv_ref[...] = dinv[:, None]


def _tc_scale(z, cnt):
    return pl.pallas_call(
        _scale_body,
        grid=(NP // R,),
        in_specs=[
            pl.BlockSpec((R, F), lambda i: (i, _z(i))),
            pl.BlockSpec((NC, R, F), lambda i: (_z(i), i, _z(i))),
        ],
        out_specs=[
            pl.BlockSpec((R, F), lambda i: (i, _z(i))),
            pl.BlockSpec((R, 1), lambda i: (i, _z(i))),
        ],
        out_shape=[
            jax.ShapeDtypeStruct((NP, F), f32),
            jax.ShapeDtypeStruct((NP, 1), f32),
        ],
    )(z, cnt)


def _valid_rows():
    rows = (lax.broadcasted_iota(jnp.int32, (R, 1), 0)
            + pl.program_id(0) * jnp.int32(R))
    return rows < jnp.int32(N)


def _comb_body_noprev(s_ref, y_ref, dinv_ref, b_ref, w_ref, h_ref, yn_ref):
    sarr = s_ref[...]
    dinv = dinv_ref[...]
    h = jnp.maximum(dinv * (sarr[0] + sarr[1] + y_ref[...]) + b_ref[...], 0.0)
    h_ref[...] = h
    yn = jnp.dot(h, w_ref[...], preferred_element_type=f32,
                 precision=PREC) * dinv
    yn_ref[...] = jnp.where(_valid_rows(), yn, 0.0)


def _comb_body_prev(s_ref, y_ref, dinv_ref, b_ref, w_ref, prev_ref,
                    h_ref, yn_ref):
    sarr = s_ref[...]
    dinv = dinv_ref[...]
    h = jnp.maximum(dinv * (sarr[0] + sarr[1] + y_ref[...]) + b_ref[...], 0.0)
    h = h + prev_ref[...]
    h_ref[...] = h
    yn = jnp.dot(h, w_ref[...], preferred_element_type=f32,
                 precision=PREC) * dinv
    yn_ref[...] = jnp.where(_valid_rows(), yn, 0.0)


def _tc_comb(s, y, dinv, b, w, prev=None):
    in_specs = [
        pl.BlockSpec((NC, R, F), lambda i: (_z(i), i, _z(i))),
        pl.BlockSpec((R, F), lambda i: (i, _z(i))),
        pl.BlockSpec((R, 1), lambda i: (i, _z(i))),
        pl.BlockSpec((1, F), lambda i: (_z(i), _z(i))),
        pl.BlockSpec((F, F), lambda i: (_z(i), _z(i))),
    ]
    args = [s, y, dinv, b, w]
    body = _comb_body_noprev
    if prev is not None:
        in_specs.append(pl.BlockSpec((R, F), lambda i: (i, _z(i))))
        args.append(prev)
        body = _comb_body_prev
    return pl.pallas_call(
        body,
        grid=(NP // R,),
        in_specs=in_specs,
        out_specs=[
            pl.BlockSpec((R, F), lambda i: (i, _z(i))),
            pl.BlockSpec((R, F), lambda i: (i, _z(i))),
        ],
        out_shape=[
            jax.ShapeDtypeStruct((NP, F), f32),
            jax.ShapeDtypeStruct((NP, F), f32),
        ],
    )(*args)


def _final_body(s_ref, y_ref, dinv_ref, b_ref, prev_ref, batch_ref,
                lw1_ref, lb1_ref, lw2_ref, lb2_ref, out_ref, sums, cnts):
    i = pl.program_id(0)

    @pl.when(i == 0)
    def _():
        sums[...] = jnp.zeros_like(sums)
        cnts[...] = jnp.zeros_like(cnts)

    sarr = s_ref[...]
    dinv = dinv_ref[...]
    h = jnp.maximum(dinv * (sarr[0] + sarr[1] + y_ref[...]) + b_ref[...], 0.0)
    h = h + prev_ref[...]
    onehot = (batch_ref[...] ==
              lax.broadcasted_iota(jnp.int32, (1, G), 1)).astype(f32)
    sums[...] += lax.dot_general(onehot, h, (((0,), (0,)), ((), ())),
                                 preferred_element_type=f32, precision=PREC)
    cnts[...] += jnp.sum(onehot, axis=0)[:, None]

    @pl.when(i == NP // R - 1)
    def _():
        g = sums[...] / jnp.maximum(cnts[...], 1.0)
        a = jnp.maximum(jnp.dot(g, lw1_ref[...], preferred_element_type=f32,
                                precision=PREC) + lb1_ref[...], 0.0)
        out_ref[...] = jnp.dot(a, lw2_ref[...], preferred_element_type=f32,
                               precision=PREC) + lb2_ref[...]


def _tc_final(s, y, dinv, b, prev, batch2d, lw1, lb1, lw2, lb2):
    return pl.pallas_call(
        _final_body,
        grid=(NP // R,),
        in_specs=[
            pl.BlockSpec((NC, R, F), lambda i: (_z(i), i, _z(i))),
            pl.BlockSpec((R, F), lambda i: (i, _z(i))),
            pl.BlockSpec((R, 1), lambda i: (i, _z(i))),
            pl.BlockSpec((1, F), lambda i: (_z(i), _z(i))),
            pl.BlockSpec((R, F), lambda i: (i, _z(i))),
            pl.BlockSpec((R, 1), lambda i: (i, _z(i))),
            pl.BlockSpec((F, F), lambda i: (_z(i), _z(i))),
            pl.BlockSpec((1, F), lambda i: (_z(i), _z(i))),
            pl.BlockSpec((F, F), lambda i: (_z(i), _z(i))),
            pl.BlockSpec((1, F), lambda i: (_z(i), _z(i))),
        ],
        out_specs=pl.BlockSpec((G, F), lambda i: (_z(i), _z(i))),
        out_shape=jax.ShapeDtypeStruct((G, F), f32),
        scratch_shapes=[
            pltpu.VMEM((G, F), f32),
            pltpu.VMEM((G, F), f32),
        ],
    )(s, y, dinv, b, prev, batch2d, lw1, lb1, lw2, lb2)


def kernel(x, edge_index, batch, W1, b1, W2, b2, W3, b3, LW1, Lb1, LW2, Lb2):
    x = x.astype(f32)
    src = edge_index[0].astype(jnp.int32)
    dst = edge_index[1].astype(jnp.int32)
    pad = N + jnp.arange(EPAD - E, dtype=jnp.int32) % jnp.int32(NP - N)
    src_p = jnp.concatenate([src, pad]).reshape(NW, NB, EB)
    dst_p = jnp.concatenate([dst, pad]).reshape(NW, NB, EB)
    batch_p = jnp.concatenate(
        [batch.astype(jnp.int32), jnp.full((NP - N,), G, jnp.int32)]
    ).reshape(NP, 1)
    x_p = jnp.pad(x, ((0, NP - N), (0, 0)))

    zeros_acc = jnp.zeros((RPT, F), f32)

    z1 = _tc_z(x_p, W1.astype(f32))
    cnt = _sc_hist(dst_p, zeros_acc)
    y1, dinv = _tc_scale(z1, cnt)

    s1 = _sc_edge(y1, src_p, dst_p, zeros_acc)
    h1, y2 = _tc_comb(s1, y1, dinv, b1.reshape(1, F).astype(f32),
                      W2.astype(f32))

    s2 = _sc_edge(y2, src_p, dst_p, zeros_acc)
    h2, y3 = _tc_comb(s2, y2, dinv, b2.reshape(1, F).astype(f32),
                      W3.astype(f32), prev=h1)

    s3 = _sc_edge(y3, src_p, dst_p, zeros_acc)
    out = _tc_final(s3, y3, dinv, b3.reshape(1, F).astype(f32), h2,
                    batch_p, LW1.astype(f32), Lb1.reshape(1, F).astype(f32),
                    LW2.astype(f32), Lb2.reshape(1, F).astype(f32))
    return out.astype(jnp.float64)
